# Initial kernel scaffold; baseline (speedup 1.0000x reference)
import jax
import jax.numpy as jnp
from jax.experimental import pallas as pl


def kernel(x, adj, Wl, bl, Wr, br, att, bias, Wp, bp, gamma, beta):
    raise NotImplementedError("write your pallas kernel here")


# speedup vs baseline: 3.3458x; 3.3458x over previous
import jax
import jax.numpy as jnp
from jax.experimental import pallas as pl
from jax.experimental.pallas import tpu as pltpu

NEG_SLOPE = 0.2
NEG_BIG = -1e30


def _body(x_ref, src_ref, dst_ref, WlT_ref, bl_ref, WrT_ref, br_ref,
          att_ref, bias_ref, WpT_ref, bp_ref, gamma_ref, beta_ref,
          out_ref, D_scr, S_scr):
    E = src_ref.shape[0]
    N = x_ref.shape[1]
    Co = att_ref.shape[1] // 2

    @pl.when(pl.program_id(0) == 0)
    def _build_onehot():
        n_iota = jax.lax.broadcasted_iota(jnp.int32, (E, N), 1)
        D_scr[...] = (dst_ref[...] == n_iota).astype(jnp.float32)
        S_scr[...] = (src_ref[...] == n_iota).astype(jnp.float32)

    xs = x_ref[0]
    x_l = jnp.dot(xs, WlT_ref[...], preferred_element_type=jnp.float32) + bl_ref[...]
    x_r = jnp.dot(xs, WrT_ref[...], preferred_element_type=jnp.float32) + br_ref[...]

    D = D_scr[...]
    S = S_scr[...]
    x_j = jnp.dot(S, x_l, preferred_element_type=jnp.float32)
    x_i = jnp.dot(D, x_r, preferred_element_type=jnp.float32)
    z = x_i + x_j
    z = jnp.where(z >= 0, z, NEG_SLOPE * z)
    za = z * att_ref[...]

    w_heads = []
    for h in range(2):
        ah = jnp.sum(za[:, h * Co:(h + 1) * Co], axis=1, keepdims=True)
        mh = jnp.where(D > 0, ah, NEG_BIG)
        amax = jnp.max(mh, axis=0, keepdims=True)
        amax = jnp.where(amax > NEG_BIG * 0.5, amax, 0.0)
        amax_dst = jnp.sum(D * amax, axis=1, keepdims=True)
        ex = jnp.exp(ah - amax_dst)
        ssum = jnp.sum(D * ex, axis=0, keepdims=True)
        ssum_dst = jnp.sum(D * ssum, axis=1, keepdims=True)
        w_heads.append(ex / (ssum_dst + 1e-16))
    w_full = jnp.concatenate(
        [jnp.broadcast_to(w_heads[0], (E, Co)),
         jnp.broadcast_to(w_heads[1], (E, Co))], axis=1)

    msg = x_j * w_full
    agg = jax.lax.dot_general(D, msg, (((0,), (0,)), ((), ())),
                              preferred_element_type=jnp.float32)
    out = jnp.maximum(agg + bias_ref[...], 0.0)
    y = jnp.dot(out, WpT_ref[...], preferred_element_type=jnp.float32) + bp_ref[...] + xs
    mu = jnp.mean(y, axis=1, keepdims=True)
    var = jnp.mean((y - mu) ** 2, axis=1, keepdims=True)
    xn = (y - mu) * jax.lax.rsqrt(var + 1e-5)
    out_ref[0] = jnp.maximum(xn * gamma_ref[...] + beta_ref[...], 0.0)


def kernel(x, adj, Wl, bl, Wr, br, att, bias, Wp, bp, gamma, beta):
    B, T, N, C = x.shape
    H, Co = att.shape
    E = adj.shape[1]
    BT = B * T

    x3 = x.reshape(BT, N, C)
    src2 = adj[0].reshape(E, 1)
    dst2 = adj[1].reshape(E, 1)

    full = lambda *shape: pl.BlockSpec(shape, lambda i: (0,) * len(shape))
    out = pl.pallas_call(
        _body,
        grid=(BT,),
        in_specs=[
            pl.BlockSpec((1, N, C), lambda i: (i, 0, 0)),
            full(E, 1), full(E, 1),
            full(C, H * Co), full(1, H * Co),
            full(C, H * Co), full(1, H * Co),
            full(1, H * Co), full(1, H * Co),
            full(H * Co, C), full(1, C),
            full(1, C), full(1, C),
        ],
        out_specs=pl.BlockSpec((1, N, C), lambda i: (i, 0, 0)),
        out_shape=jax.ShapeDtypeStruct((BT, N, C), jnp.float32),
        scratch_shapes=[pltpu.VMEM((E, N), jnp.float32),
                        pltpu.VMEM((E, N), jnp.float32)],
        compiler_params=pltpu.CompilerParams(
            dimension_semantics=("arbitrary",)),
    )(x3, src2, dst2,
      Wl.T, bl.reshape(1, H * Co), Wr.T, br.reshape(1, H * Co),
      att.reshape(1, H * Co), bias.reshape(1, H * Co),
      Wp.T, bp.reshape(1, C), gamma.reshape(1, C), beta.reshape(1, C))
    return out.reshape(B, T, N, C)
